# baseline (speedup 1.0000x reference)
import functools

import jax
import jax.numpy as jnp
from jax import lax
from jax.experimental import pallas as pl
from jax.experimental.pallas import tpu as pltpu
from jax.experimental.pallas import tpu_sc as plsc

N = 10000
E = 160000
IN_DIM = 256
HID = 256
HID2 = 2 * HID
HALF = HID // 2
NS = 16
BLK = 1000
EPS_BN = 1e-5

EDGES_PER_SUB = E // NS
EDGE_BATCH = 80
NUM_BATCHES = EDGES_PER_SUB // EDGE_BATCH
ROWS_PER_SUB = 624
ROWS_TAIL = N - NS * ROWS_PER_SUB
FCHUNKS = HALF // 16


EBLK = 2000


def _ee_body(a_ref, we1_ref, be1_ref, we2_ref, be2_ref, outl_ref, outr_ref):
    t = jnp.maximum(a_ref[...] * we1_ref[...] + be1_ref[...], 0.0)
    y = jnp.dot(t, we2_ref[...], preferred_element_type=jnp.float32) + be2_ref[...]
    outl_ref[...] = y[:, :HALF]
    outr_ref[...] = y[:, HALF:]


def _ee(a2d, we1, be1, we2, be2):
    return pl.pallas_call(
        _ee_body,
        grid=(E // EBLK,),
        in_specs=[
            pl.BlockSpec((EBLK, 1), lambda i: (i, 0)),
            pl.BlockSpec((1, HID), lambda i: (0, 0)),
            pl.BlockSpec((1, HID), lambda i: (0, 0)),
            pl.BlockSpec((HID, HID), lambda i: (0, 0)),
            pl.BlockSpec((1, HID), lambda i: (0, 0)),
        ],
        out_specs=[
            pl.BlockSpec((EBLK, HALF), lambda i: (i, 0)),
            pl.BlockSpec((EBLK, HALF), lambda i: (i, 0)),
        ],
        out_shape=[jax.ShapeDtypeStruct((E, HALF), jnp.float32)] * 2,
    )(a2d, we1, be1, we2, be2)


def _enc_body(x_ref, w1_ref, b1_ref, w2_ref, b2_ref, outl_ref, outr_ref):
    h = jnp.maximum(
        jnp.dot(x_ref[...], w1_ref[...], preferred_element_type=jnp.float32)
        + b1_ref[...], 0.0)
    y = jnp.dot(h, w2_ref[...], preferred_element_type=jnp.float32) + b2_ref[...]
    outl_ref[...] = y[:, :HALF]
    outr_ref[...] = y[:, HALF:]


def _encoder(x, w1, b1, w2, b2):
    return pl.pallas_call(
        _enc_body,
        grid=(N // BLK,),
        in_specs=[
            pl.BlockSpec((BLK, IN_DIM), lambda i: (i, 0)),
            pl.BlockSpec((IN_DIM, HID), lambda i: (0, 0)),
            pl.BlockSpec((1, HID), lambda i: (0, 0)),
            pl.BlockSpec((HID, HID), lambda i: (0, 0)),
            pl.BlockSpec((1, HID), lambda i: (0, 0)),
        ],
        out_specs=[
            pl.BlockSpec((BLK, HALF), lambda i: (i, 0)),
            pl.BlockSpec((BLK, HALF), lambda i: (i, 0)),
        ],
        out_shape=[jax.ShapeDtypeStruct((N, HALF), jnp.float32)] * 2,
    )(x, w1, b1, w2, b2)


def _mm1_body(aggl_ref, aggr_ref, hl_ref, hr_ref, eps_ref, w1_ref, b1_ref,
              u_ref, sums_ref, acc_ref):
    i = pl.program_id(0)
    t = jnp.concatenate([aggl_ref[...], aggr_ref[...]], axis=1)
    hcat = jnp.concatenate([hl_ref[...], hr_ref[...]], axis=1)
    t = t + eps_ref[0, 0] * hcat
    u = jnp.dot(t, w1_ref[...], preferred_element_type=jnp.float32) + b1_ref[...]
    u_ref[...] = u
    part = jnp.concatenate(
        [jnp.sum(u, axis=0, keepdims=True),
         jnp.sum(u * u, axis=0, keepdims=True)], axis=0)

    @pl.when(i == 0)
    def _():
        acc_ref[...] = part

    @pl.when(i > 0)
    def _():
        acc_ref[...] += part

    @pl.when(i == pl.num_programs(0) - 1)
    def _():
        sums_ref[...] = acc_ref[...]


def _mm1(aggl, aggr, hl, hr, eps, w1, b1):
    return pl.pallas_call(
        _mm1_body,
        grid=(N // BLK,),
        in_specs=[
            pl.BlockSpec((BLK, HALF), lambda i: (i, 0)),
            pl.BlockSpec((BLK, HALF), lambda i: (i, 0)),
            pl.BlockSpec((BLK, HALF), lambda i: (i, 0)),
            pl.BlockSpec((BLK, HALF), lambda i: (i, 0)),
            pl.BlockSpec((1, 1), lambda i: (0, 0)),
            pl.BlockSpec((HID, HID2), lambda i: (0, 0)),
            pl.BlockSpec((1, HID2), lambda i: (0, 0)),
        ],
        out_specs=[
            pl.BlockSpec((BLK, HID2), lambda i: (i, 0)),
            pl.BlockSpec((2, HID2), lambda i: (0, 0)),
        ],
        out_shape=[
            jax.ShapeDtypeStruct((N, HID2), jnp.float32),
            jax.ShapeDtypeStruct((2, HID2), jnp.float32),
        ],
        scratch_shapes=[pltpu.VMEM((2, HID2), jnp.float32)],
    )(aggl, aggr, hl, hr, eps, w1, b1)


def _mm2_common(u_ref, sums_ref, g_ref, beta_ref, w2_ref, b2_ref):
    m = sums_ref[0:1] * (1.0 / N)
    var = sums_ref[1:2] * (1.0 / N) - m * m
    rstd = lax.rsqrt(var + EPS_BN)
    hn = jnp.maximum((u_ref[...] - m) * (rstd * g_ref[...]) + beta_ref[...], 0.0)
    return jnp.dot(hn, w2_ref[...], preferred_element_type=jnp.float32) + b2_ref[...]


def _mm2_relu_body(u_ref, sums_ref, g_ref, beta_ref, w2_ref, b2_ref,
                   outl_ref, outr_ref):
    y = jnp.maximum(_mm2_common(u_ref, sums_ref, g_ref, beta_ref, w2_ref, b2_ref), 0.0)
    outl_ref[...] = y[:, :HALF]
    outr_ref[...] = y[:, HALF:]


def _mm2_plain_body(u_ref, sums_ref, g_ref, beta_ref, w2_ref, b2_ref, out_ref):
    out_ref[...] = _mm2_common(u_ref, sums_ref, g_ref, beta_ref, w2_ref, b2_ref)


def _mm2_z_body(u_ref, sums_ref, g_ref, beta_ref, w2_ref, b2_ref,
                mean_ref, noise_ref, out_ref):
    y = _mm2_common(u_ref, sums_ref, g_ref, beta_ref, w2_ref, b2_ref)
    out_ref[...] = mean_ref[...] + noise_ref[...] * jnp.exp(y)


_MM2_IN_SPECS = [
    pl.BlockSpec((BLK, HID2), lambda i: (i, 0)),
    pl.BlockSpec((2, HID2), lambda i: (0, 0)),
    pl.BlockSpec((1, HID2), lambda i: (0, 0)),
    pl.BlockSpec((1, HID2), lambda i: (0, 0)),
    pl.BlockSpec((HID2, HID), lambda i: (0, 0)),
    pl.BlockSpec((1, HID), lambda i: (0, 0)),
]


def _mm2_relu(u, sums, g, beta, w2, b2):
    return pl.pallas_call(
        _mm2_relu_body,
        grid=(N // BLK,),
        in_specs=_MM2_IN_SPECS,
        out_specs=[
            pl.BlockSpec((BLK, HALF), lambda i: (i, 0)),
            pl.BlockSpec((BLK, HALF), lambda i: (i, 0)),
        ],
        out_shape=[jax.ShapeDtypeStruct((N, HALF), jnp.float32)] * 2,
    )(u, sums, g, beta, w2, b2)


def _mm2_plain(u, sums, g, beta, w2, b2):
    return pl.pallas_call(
        _mm2_plain_body,
        grid=(N // BLK,),
        in_specs=_MM2_IN_SPECS,
        out_specs=pl.BlockSpec((BLK, HID), lambda i: (i, 0)),
        out_shape=jax.ShapeDtypeStruct((N, HID), jnp.float32),
    )(u, sums, g, beta, w2, b2)


def _mm2_z(u, sums, g, beta, w2, b2, mean, noise):
    return pl.pallas_call(
        _mm2_z_body,
        grid=(N // BLK,),
        in_specs=_MM2_IN_SPECS + [
            pl.BlockSpec((BLK, HID), lambda i: (i, 0)),
            pl.BlockSpec((BLK, HID), lambda i: (i, 0)),
        ],
        out_specs=pl.BlockSpec((BLK, HID), lambda i: (i, 0)),
        out_shape=jax.ShapeDtypeStruct((N, HID), jnp.float32),
    )(u, sums, g, beta, w2, b2, mean, noise)


def _edge_body(hl_ref, hr_ref, src_ref, dst_ref, eel_ref, eer_ref,
               outl_ref, outr_ref,
               shared, srcv, dstv, rows, eeb, msg, sem):
    c = lax.axis_index("c")
    s = lax.axis_index("s")
    row0 = s * ROWS_PER_SUB

    def run(h_half, ee_half, out_half):
        pltpu.sync_copy(h_half.at[pl.ds(row0, ROWS_PER_SUB)],
                        shared.at[pl.ds(row0, ROWS_PER_SUB)])

        @pl.when(s == NS - 1)
        def _():
            pltpu.sync_copy(h_half.at[pl.ds(NS * ROWS_PER_SUB, ROWS_TAIL)],
                            shared.at[pl.ds(NS * ROWS_PER_SUB, ROWS_TAIL)])

        plsc.subcore_barrier()

        e0 = s * EDGES_PER_SUB

        def batch_body(k, carry):
            base = e0 + k * EDGE_BATCH
            pltpu.sync_copy(src_ref.at[pl.ds(base, EDGE_BATCH)], srcv)
            pltpu.sync_copy(dst_ref.at[pl.ds(base, EDGE_BATCH)], dstv)
            pltpu.sync_copy(ee_half.at[pl.ds(base, EDGE_BATCH)], eeb)
            pltpu.async_copy(h_half.at[srcv], rows, sem).wait()

            def edge_loop(e, carry2):
                for j in range(FCHUNKS):
                    v = rows[e, pl.ds(16 * j, 16)] + eeb[e, pl.ds(16 * j, 16)]
                    msg[e, pl.ds(16 * j, 16)] = jnp.maximum(v, 0.0)
                return carry2

            lax.fori_loop(0, EDGE_BATCH, edge_loop, 0)
            pltpu.sync_copy(msg, shared.at[dstv], add=True)
            return carry

        lax.fori_loop(0, NUM_BATCHES, batch_body, 0)
        plsc.subcore_barrier()
        pltpu.sync_copy(shared.at[pl.ds(row0, ROWS_PER_SUB)],
                        out_half.at[pl.ds(row0, ROWS_PER_SUB)])

        @pl.when(s == NS - 1)
        def _():
            pltpu.sync_copy(shared.at[pl.ds(NS * ROWS_PER_SUB, ROWS_TAIL)],
                            out_half.at[pl.ds(NS * ROWS_PER_SUB, ROWS_TAIL)])

    @pl.when(c == 0)
    def _():
        run(hl_ref, eel_ref, outl_ref)

    @pl.when(c == 1)
    def _():
        run(hr_ref, eer_ref, outr_ref)


_edge_aggregate = pl.kernel(
    _edge_body,
    out_type=[jax.ShapeDtypeStruct((N, HALF), jnp.float32)] * 2,
    mesh=plsc.VectorSubcoreMesh(core_axis_name="c", subcore_axis_name="s"),
    scratch_types=[
        pltpu.VMEM_SHARED((N, HALF), jnp.float32),
        pltpu.VMEM((EDGE_BATCH,), jnp.int32),
        pltpu.VMEM((EDGE_BATCH,), jnp.int32),
        pltpu.VMEM((EDGE_BATCH, HALF), jnp.float32),
        pltpu.VMEM((EDGE_BATCH, HALF), jnp.float32),
        pltpu.VMEM((EDGE_BATCH, HALF), jnp.float32),
        pltpu.SemaphoreType.DMA,
    ],
)


def kernel(x, edge_index, edge_attr, params):
    convs = params['convs']
    src = edge_index[0]
    dst = edge_index[1]

    ees = [_ee(edge_attr, c['We1'], c['be1'][None, :], c['We2'],
               c['be2'][None, :]) for c in convs]

    h0l, h0r = _encoder(x, params['Wx1'], params['bx1'][None, :],
                        params['Wx2'], params['bx2'][None, :])

    def gin(hl, hr, conv, ee):
        aggl, aggr = _edge_aggregate(hl, hr, src, dst, ee[0], ee[1])
        eps = conv['eps'].reshape(1, 1)
        u, sums = _mm1(aggl, aggr, hl, hr, eps, conv['W1'], conv['b1'][None, :])
        return u, sums, conv

    u0, s0, c0 = gin(h0l, h0r, convs[0], ees[0])
    h1l, h1r = _mm2_relu(u0, s0, c0['g'][None, :], c0['beta'][None, :],
                         c0['W2'], c0['b2'][None, :])

    u1, s1, c1 = gin(h1l, h1r, convs[1], ees[1])
    mean = _mm2_plain(u1, s1, c1['g'][None, :], c1['beta'][None, :],
                      c1['W2'], c1['b2'][None, :])

    u2, s2, c2 = gin(h1l, h1r, convs[2], ees[2])
    noise = jax.random.normal(jax.random.key(42), (N, HID), dtype=jnp.float32)
    z = _mm2_z(u2, s2, c2['g'][None, :], c2['beta'][None, :],
               c2['W2'], c2['b2'][None, :], mean, noise)
    return z


# speedup vs baseline: 1.6360x; 1.6360x over previous
import functools

import jax
import jax.numpy as jnp
from jax import lax
from jax.experimental import pallas as pl
from jax.experimental.pallas import tpu as pltpu
from jax.experimental.pallas import tpu_sc as plsc

N = 10000
E = 160000
IN_DIM = 256
HID = 256
HID2 = 2 * HID
HALF = HID // 2
NS = 16
BLK = 1000
EPS_BN = 1e-5

EDGES_PER_SUB = E // NS
EDGE_BATCH = 80
NUM_BATCHES = EDGES_PER_SUB // EDGE_BATCH
ROWS_PER_SUB = 624
ROWS_TAIL = N - NS * ROWS_PER_SUB
FCHUNKS = HALF // 16


EBLK = 2000


def _ee_body(a_ref, we1_ref, be1_ref, we2_ref, be2_ref, outl_ref, outr_ref):
    t = jnp.maximum(a_ref[...] * we1_ref[...] + be1_ref[...], 0.0)
    y = jnp.dot(t, we2_ref[...], preferred_element_type=jnp.float32) + be2_ref[...]
    outl_ref[...] = y[:, :HALF]
    outr_ref[...] = y[:, HALF:]


def _ee(a2d, we1, be1, we2, be2):
    return pl.pallas_call(
        _ee_body,
        grid=(E // EBLK,),
        in_specs=[
            pl.BlockSpec((EBLK, 1), lambda i: (i, 0)),
            pl.BlockSpec((1, HID), lambda i: (0, 0)),
            pl.BlockSpec((1, HID), lambda i: (0, 0)),
            pl.BlockSpec((HID, HID), lambda i: (0, 0)),
            pl.BlockSpec((1, HID), lambda i: (0, 0)),
        ],
        out_specs=[
            pl.BlockSpec((EBLK, HALF), lambda i: (i, 0)),
            pl.BlockSpec((EBLK, HALF), lambda i: (i, 0)),
        ],
        out_shape=[jax.ShapeDtypeStruct((E, HALF), jnp.float32)] * 2,
    )(a2d, we1, be1, we2, be2)


def _enc_body(x_ref, w1_ref, b1_ref, w2_ref, b2_ref, outl_ref, outr_ref):
    h = jnp.maximum(
        jnp.dot(x_ref[...], w1_ref[...], preferred_element_type=jnp.float32)
        + b1_ref[...], 0.0)
    y = jnp.dot(h, w2_ref[...], preferred_element_type=jnp.float32) + b2_ref[...]
    outl_ref[...] = y[:, :HALF]
    outr_ref[...] = y[:, HALF:]


def _encoder(x, w1, b1, w2, b2):
    return pl.pallas_call(
        _enc_body,
        grid=(N // BLK,),
        in_specs=[
            pl.BlockSpec((BLK, IN_DIM), lambda i: (i, 0)),
            pl.BlockSpec((IN_DIM, HID), lambda i: (0, 0)),
            pl.BlockSpec((1, HID), lambda i: (0, 0)),
            pl.BlockSpec((HID, HID), lambda i: (0, 0)),
            pl.BlockSpec((1, HID), lambda i: (0, 0)),
        ],
        out_specs=[
            pl.BlockSpec((BLK, HALF), lambda i: (i, 0)),
            pl.BlockSpec((BLK, HALF), lambda i: (i, 0)),
        ],
        out_shape=[jax.ShapeDtypeStruct((N, HALF), jnp.float32)] * 2,
    )(x, w1, b1, w2, b2)


def _mm1_body(aggl_ref, aggr_ref, hl_ref, hr_ref, eps_ref, w1_ref, b1_ref,
              u_ref, sums_ref, acc_ref):
    i = pl.program_id(0)
    t = jnp.concatenate([aggl_ref[...], aggr_ref[...]], axis=1)
    hcat = jnp.concatenate([hl_ref[...], hr_ref[...]], axis=1)
    t = t + eps_ref[0, 0] * hcat
    u = jnp.dot(t, w1_ref[...], preferred_element_type=jnp.float32) + b1_ref[...]
    u_ref[...] = u
    part = jnp.concatenate(
        [jnp.sum(u, axis=0, keepdims=True),
         jnp.sum(u * u, axis=0, keepdims=True)], axis=0)

    @pl.when(i == 0)
    def _():
        acc_ref[...] = part

    @pl.when(i > 0)
    def _():
        acc_ref[...] += part

    @pl.when(i == pl.num_programs(0) - 1)
    def _():
        sums_ref[...] = acc_ref[...]


def _mm1(aggl, aggr, hl, hr, eps, w1, b1):
    return pl.pallas_call(
        _mm1_body,
        grid=(N // BLK,),
        in_specs=[
            pl.BlockSpec((BLK, HALF), lambda i: (i, 0)),
            pl.BlockSpec((BLK, HALF), lambda i: (i, 0)),
            pl.BlockSpec((BLK, HALF), lambda i: (i, 0)),
            pl.BlockSpec((BLK, HALF), lambda i: (i, 0)),
            pl.BlockSpec((1, 1), lambda i: (0, 0)),
            pl.BlockSpec((HID, HID2), lambda i: (0, 0)),
            pl.BlockSpec((1, HID2), lambda i: (0, 0)),
        ],
        out_specs=[
            pl.BlockSpec((BLK, HID2), lambda i: (i, 0)),
            pl.BlockSpec((2, HID2), lambda i: (0, 0)),
        ],
        out_shape=[
            jax.ShapeDtypeStruct((N, HID2), jnp.float32),
            jax.ShapeDtypeStruct((2, HID2), jnp.float32),
        ],
        scratch_shapes=[pltpu.VMEM((2, HID2), jnp.float32)],
    )(aggl, aggr, hl, hr, eps, w1, b1)


def _mm2_common(u_ref, sums_ref, g_ref, beta_ref, w2_ref, b2_ref):
    m = sums_ref[0:1] * (1.0 / N)
    var = sums_ref[1:2] * (1.0 / N) - m * m
    rstd = lax.rsqrt(var + EPS_BN)
    hn = jnp.maximum((u_ref[...] - m) * (rstd * g_ref[...]) + beta_ref[...], 0.0)
    return jnp.dot(hn, w2_ref[...], preferred_element_type=jnp.float32) + b2_ref[...]


def _mm2_relu_body(u_ref, sums_ref, g_ref, beta_ref, w2_ref, b2_ref,
                   outl_ref, outr_ref):
    y = jnp.maximum(_mm2_common(u_ref, sums_ref, g_ref, beta_ref, w2_ref, b2_ref), 0.0)
    outl_ref[...] = y[:, :HALF]
    outr_ref[...] = y[:, HALF:]


def _mm2_plain_body(u_ref, sums_ref, g_ref, beta_ref, w2_ref, b2_ref, out_ref):
    out_ref[...] = _mm2_common(u_ref, sums_ref, g_ref, beta_ref, w2_ref, b2_ref)


def _mm2_z_body(u_ref, sums_ref, g_ref, beta_ref, w2_ref, b2_ref,
                mean_ref, noise_ref, out_ref):
    y = _mm2_common(u_ref, sums_ref, g_ref, beta_ref, w2_ref, b2_ref)
    out_ref[...] = mean_ref[...] + noise_ref[...] * jnp.exp(y)


_MM2_IN_SPECS = [
    pl.BlockSpec((BLK, HID2), lambda i: (i, 0)),
    pl.BlockSpec((2, HID2), lambda i: (0, 0)),
    pl.BlockSpec((1, HID2), lambda i: (0, 0)),
    pl.BlockSpec((1, HID2), lambda i: (0, 0)),
    pl.BlockSpec((HID2, HID), lambda i: (0, 0)),
    pl.BlockSpec((1, HID), lambda i: (0, 0)),
]


def _mm2_relu(u, sums, g, beta, w2, b2):
    return pl.pallas_call(
        _mm2_relu_body,
        grid=(N // BLK,),
        in_specs=_MM2_IN_SPECS,
        out_specs=[
            pl.BlockSpec((BLK, HALF), lambda i: (i, 0)),
            pl.BlockSpec((BLK, HALF), lambda i: (i, 0)),
        ],
        out_shape=[jax.ShapeDtypeStruct((N, HALF), jnp.float32)] * 2,
    )(u, sums, g, beta, w2, b2)


def _mm2_plain(u, sums, g, beta, w2, b2):
    return pl.pallas_call(
        _mm2_plain_body,
        grid=(N // BLK,),
        in_specs=_MM2_IN_SPECS,
        out_specs=pl.BlockSpec((BLK, HID), lambda i: (i, 0)),
        out_shape=jax.ShapeDtypeStruct((N, HID), jnp.float32),
    )(u, sums, g, beta, w2, b2)


def _mm2_z(u, sums, g, beta, w2, b2, mean, noise):
    return pl.pallas_call(
        _mm2_z_body,
        grid=(N // BLK,),
        in_specs=_MM2_IN_SPECS + [
            pl.BlockSpec((BLK, HID), lambda i: (i, 0)),
            pl.BlockSpec((BLK, HID), lambda i: (i, 0)),
        ],
        out_specs=pl.BlockSpec((BLK, HID), lambda i: (i, 0)),
        out_shape=jax.ShapeDtypeStruct((N, HID), jnp.float32),
    )(u, sums, g, beta, w2, b2, mean, noise)


def _edge_body(hl_ref, hr_ref, src_ref, dst_ref, eel_ref, eer_ref,
               outl_ref, outr_ref,
               shared, srcv0, srcv1, dstv0, dstv1, eeb0, eeb1,
               rows0, rows1,
               semi0, semi1, semg0, semg1, semd0, semd1, sems0, sems1):
    c = lax.axis_index("c")
    s = lax.axis_index("s")
    row0 = s * ROWS_PER_SUB
    e0 = s * EDGES_PER_SUB

    srcv = (srcv0, srcv1)
    dstv = (dstv0, dstv1)
    eeb = (eeb0, eeb1)
    rows = (rows0, rows1)
    semi = (semi0, semi1)
    semg = (semg0, semg1)
    semd = (semd0, semd1)
    sems = (sems0, sems1)

    def run(h_half, ee_half, out_half):
        pltpu.sync_copy(h_half.at[pl.ds(row0, ROWS_PER_SUB)],
                        shared.at[pl.ds(row0, ROWS_PER_SUB)])

        @pl.when(s == NS - 1)
        def _():
            pltpu.sync_copy(h_half.at[pl.ds(NS * ROWS_PER_SUB, ROWS_TAIL)],
                            shared.at[pl.ds(NS * ROWS_PER_SUB, ROWS_TAIL)])

        plsc.subcore_barrier()

        def ebase(k):
            return e0 + k * EDGE_BATCH

        def issue_loads(k, b):
            pltpu.async_copy(src_ref.at[pl.ds(ebase(k), EDGE_BATCH)], srcv[b], semi[b])
            pltpu.async_copy(ee_half.at[pl.ds(ebase(k), EDGE_BATCH)], eeb[b], semi[b])

        def wait_loads(k, b):
            pltpu.make_async_copy(src_ref.at[pl.ds(ebase(k), EDGE_BATCH)], srcv[b], semi[b]).wait()
            pltpu.make_async_copy(ee_half.at[pl.ds(ebase(k), EDGE_BATCH)], eeb[b], semi[b]).wait()

        def issue_gather(b):
            pltpu.async_copy(h_half.at[srcv[b]], rows[b], semg[b])

        def wait_gather(b):
            pltpu.make_async_copy(h_half.at[srcv[b]], rows[b], semg[b]).wait()

        def issue_dst(k, b):
            pltpu.async_copy(dst_ref.at[pl.ds(ebase(k), EDGE_BATCH)], dstv[b], semd[b])

        def wait_dst(k, b):
            pltpu.make_async_copy(dst_ref.at[pl.ds(ebase(k), EDGE_BATCH)], dstv[b], semd[b]).wait()

        def issue_scatter(b):
            pltpu.async_copy(rows[b], shared.at[dstv[b]], sems[b], add=True)

        def wait_scatter(b):
            pltpu.make_async_copy(rows[b], shared.at[dstv[b]], sems[b]).wait()

        def compute(b):
            def edge_loop(e, carry2):
                for j in range(FCHUNKS):
                    v = rows[b][e, pl.ds(16 * j, 16)] + eeb[b][e, pl.ds(16 * j, 16)]
                    rows[b][e, pl.ds(16 * j, 16)] = jnp.maximum(v, 0.0)
                return carry2
            lax.fori_loop(0, EDGE_BATCH, edge_loop, 0)

        issue_loads(0, 0)
        issue_loads(1, 1)
        wait_loads(0, 0)
        issue_gather(0)
        issue_dst(0, 0)

        def pair_body(i, carry):
            k0 = 2 * i
            k1 = k0 + 1
            wait_gather(0)
            wait_dst(k0, 0)
            compute(0)
            issue_scatter(0)
            issue_loads(k0 + 2, 0)
            wait_loads(k1, 1)

            @pl.when(i > 0)
            def _():
                wait_scatter(1)

            issue_gather(1)
            issue_dst(k1, 1)
            wait_gather(1)
            wait_dst(k1, 1)
            compute(1)
            issue_scatter(1)

            @pl.when(k1 + 2 < NUM_BATCHES)
            def _():
                issue_loads(k1 + 2, 1)

            wait_loads(k0 + 2, 0)
            wait_scatter(0)
            issue_gather(0)
            issue_dst(k0 + 2, 0)
            return carry

        lax.fori_loop(0, (NUM_BATCHES - 1) // 2, pair_body, 0)

        klast = NUM_BATCHES - 1
        wait_gather(0)
        wait_dst(klast, 0)
        compute(0)
        issue_scatter(0)
        wait_scatter(1)
        wait_scatter(0)

        plsc.subcore_barrier()
        pltpu.sync_copy(shared.at[pl.ds(row0, ROWS_PER_SUB)],
                        out_half.at[pl.ds(row0, ROWS_PER_SUB)])

        @pl.when(s == NS - 1)
        def _():
            pltpu.sync_copy(shared.at[pl.ds(NS * ROWS_PER_SUB, ROWS_TAIL)],
                            out_half.at[pl.ds(NS * ROWS_PER_SUB, ROWS_TAIL)])

    @pl.when(c == 0)
    def _():
        run(hl_ref, eel_ref, outl_ref)

    @pl.when(c == 1)
    def _():
        run(hr_ref, eer_ref, outr_ref)


_edge_aggregate = pl.kernel(
    _edge_body,
    out_type=[jax.ShapeDtypeStruct((N, HALF), jnp.float32)] * 2,
    mesh=plsc.VectorSubcoreMesh(core_axis_name="c", subcore_axis_name="s"),
    scratch_types=[
        pltpu.VMEM_SHARED((N, HALF), jnp.float32),
        pltpu.VMEM((EDGE_BATCH,), jnp.int32),
        pltpu.VMEM((EDGE_BATCH,), jnp.int32),
        pltpu.VMEM((EDGE_BATCH,), jnp.int32),
        pltpu.VMEM((EDGE_BATCH,), jnp.int32),
        pltpu.VMEM((EDGE_BATCH, HALF), jnp.float32),
        pltpu.VMEM((EDGE_BATCH, HALF), jnp.float32),
        pltpu.VMEM((EDGE_BATCH, HALF), jnp.float32),
        pltpu.VMEM((EDGE_BATCH, HALF), jnp.float32),
        pltpu.SemaphoreType.DMA,
        pltpu.SemaphoreType.DMA,
        pltpu.SemaphoreType.DMA,
        pltpu.SemaphoreType.DMA,
        pltpu.SemaphoreType.DMA,
        pltpu.SemaphoreType.DMA,
        pltpu.SemaphoreType.DMA,
        pltpu.SemaphoreType.DMA,
    ],
)


def kernel(x, edge_index, edge_attr, params):
    convs = params['convs']
    src = edge_index[0]
    dst = edge_index[1]

    ees = [_ee(edge_attr, c['We1'], c['be1'][None, :], c['We2'],
               c['be2'][None, :]) for c in convs]

    h0l, h0r = _encoder(x, params['Wx1'], params['bx1'][None, :],
                        params['Wx2'], params['bx2'][None, :])

    def gin(hl, hr, conv, ee):
        aggl, aggr = _edge_aggregate(hl, hr, src, dst, ee[0], ee[1])
        eps = conv['eps'].reshape(1, 1)
        u, sums = _mm1(aggl, aggr, hl, hr, eps, conv['W1'], conv['b1'][None, :])
        return u, sums, conv

    u0, s0, c0 = gin(h0l, h0r, convs[0], ees[0])
    h1l, h1r = _mm2_relu(u0, s0, c0['g'][None, :], c0['beta'][None, :],
                         c0['W2'], c0['b2'][None, :])

    u1, s1, c1 = gin(h1l, h1r, convs[1], ees[1])
    mean = _mm2_plain(u1, s1, c1['g'][None, :], c1['beta'][None, :],
                      c1['W2'], c1['b2'][None, :])

    u2, s2, c2 = gin(h1l, h1r, convs[2], ees[2])
    noise = jax.random.normal(jax.random.key(42), (N, HID), dtype=jnp.float32)
    z = _mm2_z(u2, s2, c2['g'][None, :], c2['beta'][None, :],
               c2['W2'], c2['b2'][None, :], mean, noise)
    return z
